# Initial kernel scaffold; baseline (speedup 1.0000x reference)
import jax
import jax.numpy as jnp
from jax.experimental import pallas as pl


def kernel(inputs, node_embeddings, W1, a_src1, a_dst1, b1, W2, a_src2, a_dst2, b2):
    raise NotImplementedError("write your pallas kernel here")


# speedup vs baseline: 1.0638x; 1.0638x over previous
import jax
import jax.numpy as jnp
from jax.experimental import pallas as pl

N = 10000
K = 20


def _gat_mine(x, src, dst, W, a_src, a_dst, b, concat):
    Hh, C = a_src.shape
    h = (x @ W).reshape(x.shape[0], Hh, C)
    alpha_s = jnp.sum(h * a_src[None, :, :], axis=-1)
    alpha_d = jnp.sum(h * a_dst[None, :, :], axis=-1)
    e = jax.nn.leaky_relu(alpha_s[src] + alpha_d[dst], negative_slope=0.2)
    ee = jnp.exp(e)
    denom = jax.ops.segment_sum(ee, dst, num_segments=N)
    att = ee / (denom[dst] + 1e-9)
    msg = h[src] * att[:, :, None]
    out = jax.ops.segment_sum(msg, dst, num_segments=N)
    if concat:
        out = out.reshape(N, Hh * C)
    else:
        out = jnp.mean(out, axis=1)
    return out + b


def _edges_mine(node_embeddings):
    nrm = jnp.sqrt(jnp.maximum(jnp.sum(node_embeddings * node_embeddings, axis=-1, keepdims=True), 1e-12))
    norm_emb = node_embeddings / nrm
    sim = norm_emb @ norm_emb.T
    _, topk_idx = jax.lax.top_k(sim, K)
    src = jnp.repeat(jnp.arange(N, dtype=jnp.int32), K)
    dst = topk_idx.reshape(-1).astype(jnp.int32)
    rng = jnp.arange(N, dtype=jnp.int32)
    src = jnp.concatenate([src, rng], axis=0)
    dst = jnp.concatenate([dst, rng], axis=0)
    return src, dst


def kernel(inputs, node_embeddings, W1, a_src1, a_dst1, b1, W2, a_src2, a_dst2, b2):
    src, dst = _edges_mine(node_embeddings)
    x = _gat_mine(inputs, src, dst, W1, a_src1, a_dst1, b1, concat=True)
    x = jax.nn.relu(x)
    out = _gat_mine(x, src, dst, W2, a_src2, a_dst2, b2, concat=False)
    return out


# speedup vs baseline: 1.2982x; 1.2203x over previous
import functools
import jax
import jax.numpy as jnp
from jax import lax
from jax.experimental import pallas as pl
from jax.experimental.pallas import tpu as pltpu

N = 10000
NP = 10240
D = 256
UNITS = 128
H1 = 4
K = 20
EMB = 16
CROW = 144
EPR = 24
BR = 128
NEG = -3e38


def _k1_body(x_ref, emb_ref, w1_ref, as1_ref, ad1_ref,
             h1aug_ref, nemb_ref, asout_ref, adout_ref):
    i = pl.program_id(0)
    rows = jax.lax.broadcasted_iota(jnp.int32, (BR, 1), 0) + i * BR
    valid = (rows < N).astype(jnp.float32)

    e = emb_ref[...]
    nrm = jnp.sqrt(jnp.maximum(jnp.sum(e * e, axis=-1, keepdims=True), 1e-12))
    nemb_ref[...] = e / nrm

    h = jnp.dot(x_ref[...], w1_ref[...],
                preferred_element_type=jnp.float32)
    h3 = h.reshape(BR, H1, UNITS)
    asv = jnp.sum(h3 * as1_ref[...][None, :, :], axis=-1)
    adv = jnp.sum(h3 * ad1_ref[...][None, :, :], axis=-1)
    zpad = jnp.zeros((BR, 8 - H1), jnp.float32)
    asout_ref[...] = jnp.concatenate([asv * valid, zpad], axis=1)
    adout_ref[...] = jnp.concatenate([adv * valid, zpad], axis=1)

    ones = valid
    zc = jnp.zeros((BR, CROW - UNITS - 1), jnp.float32)
    for g in range(H1):
        blk = jnp.concatenate([h3[:, g, :] * valid, ones, zc], axis=1)
        h1aug_ref[g, :, :] = blk


def _dense_front(x_pad, emb_pad, W1, a_src1, a_dst1):
    grid = NP // BR
    return pl.pallas_call(
        _k1_body,
        grid=(grid,),
        in_specs=[
            pl.BlockSpec((BR, D), lambda i: (i, 0)),
            pl.BlockSpec((BR, EMB), lambda i: (i, 0)),
            pl.BlockSpec((D, H1 * UNITS), lambda i: (0, 0)),
            pl.BlockSpec((H1, UNITS), lambda i: (0, 0)),
            pl.BlockSpec((H1, UNITS), lambda i: (0, 0)),
        ],
        out_specs=[
            pl.BlockSpec((H1, BR, CROW), lambda i: (0, i, 0)),
            pl.BlockSpec((BR, EMB), lambda i: (i, 0)),
            pl.BlockSpec((BR, 8), lambda i: (i, 0)),
            pl.BlockSpec((BR, 8), lambda i: (i, 0)),
        ],
        out_shape=[
            jax.ShapeDtypeStruct((H1, NP, CROW), jnp.float32),
            jax.ShapeDtypeStruct((NP, EMB), jnp.float32),
            jax.ShapeDtypeStruct((NP, 8), jnp.float32),
            jax.ShapeDtypeStruct((NP, 8), jnp.float32),
        ],
    )(x_pad, emb_pad, W1, a_src1, a_dst1)


NCHUNK = NP // 128


def _topk_body(nemb_ref, nembT_ref, dst_ref, src_ref):
    i = pl.program_id(0)
    sim = jnp.dot(nemb_ref[...], nembT_ref[...],
                  preferred_element_type=jnp.float32)
    col = jax.lax.broadcasted_iota(jnp.int32, (BR, NP), 1)
    sim = jnp.where(col < N, sim, NEG)

    lane = jax.lax.broadcasted_iota(jnp.int32, (BR, 128), 1)

    def fold(c, carry):
        m1, m2, m3, m4, c1, c2, c3, c4 = carry
        v = sim[:, c * 128:(c + 1) * 128]
        ci = jnp.full((BR, 128), c, jnp.int32)
        g1 = v > m1
        g2 = v > m2
        g3 = v > m3
        g4 = v > m4
        n1 = jnp.where(g1, v, m1)
        n2 = jnp.where(g1, m1, jnp.where(g2, v, m2))
        n3 = jnp.where(g2, m2, jnp.where(g3, v, m3))
        n4 = jnp.where(g3, m3, jnp.where(g4, v, m4))
        i1 = jnp.where(g1, ci, c1)
        i2 = jnp.where(g1, c1, jnp.where(g2, ci, c2))
        i3 = jnp.where(g2, c2, jnp.where(g3, ci, c3))
        i4 = jnp.where(g3, c3, jnp.where(g4, ci, c4))
        return n1, n2, n3, n4, i1, i2, i3, i4

    neg = jnp.full((BR, 128), NEG, jnp.float32)
    zi = jnp.zeros((BR, 128), jnp.int32)
    carry = (neg, neg, neg, neg, zi, zi, zi, zi)
    for c in range(NCHUNK):
        carry = fold(c, carry)
    m1, m2, m3, m4, c1, c2, c3, c4 = carry

    rows = jax.lax.broadcasted_iota(jnp.int32, (BR, 1), 0) + i * BR
    lane24 = jax.lax.broadcasted_iota(jnp.int32, (BR, EPR), 1)
    BIGI = jnp.int32(2 ** 30)

    def extract(t, carry):
        m1, m2, m3, m4, out = carry
        cur = jnp.maximum(jnp.maximum(m1, m2), jnp.maximum(m3, m4))
        rmax = jnp.max(cur, axis=1, keepdims=True)
        k1 = jnp.where(m1 >= rmax, c1 * 128 + lane, BIGI)
        k2 = jnp.where(m2 >= rmax, c2 * 128 + lane, BIGI)
        k3 = jnp.where(m3 >= rmax, c3 * 128 + lane, BIGI)
        k4 = jnp.where(m4 >= rmax, c4 * 128 + lane, BIGI)
        kk = jnp.minimum(jnp.minimum(k1, k2), jnp.minimum(k3, k4))
        idx = jnp.min(kk, axis=1, keepdims=True)
        out = jnp.where(lane24 == t, idx, out)
        m1 = jnp.where(k1 == idx, NEG, m1)
        m2 = jnp.where((k2 == idx) & (k1 != idx), NEG, m2)
        m3 = jnp.where((k3 == idx) & (k2 != idx) & (k1 != idx), NEG, m3)
        m4 = jnp.where((k4 == idx) & (k3 != idx) & (k2 != idx) & (k1 != idx),
                       NEG, m4)
        return m1, m2, m3, m4, out

    out0 = jnp.zeros((BR, EPR), jnp.int32)
    _, _, _, _, out = lax.fori_loop(0, K, extract, (m1, m2, m3, m4, out0))

    dst_ref[...] = jnp.where(lane24 >= K, rows, out)
    src_ref[...] = jnp.where(lane24 <= K, rows, jnp.int32(NP - 1))


def _topk_edges(nemb, nembT):
    grid = NP // BR
    return pl.pallas_call(
        _topk_body,
        grid=(grid,),
        in_specs=[
            pl.BlockSpec((BR, EMB), lambda i: (i, 0)),
            pl.BlockSpec((EMB, NP), lambda i: (0, 0)),
        ],
        out_specs=[
            pl.BlockSpec((BR, EPR), lambda i: (i, 0)),
            pl.BlockSpec((BR, EPR), lambda i: (i, 0)),
        ],
        out_shape=[
            jax.ShapeDtypeStruct((NP, EPR), jnp.int32),
            jax.ShapeDtypeStruct((NP, EPR), jnp.int32),
        ],
    )(nemb, nembT)


def _seg_aggregate(h_aug, asv, adv, dst, src, n_heads):
    accs = []
    for g in range(n_heads):
        e = asv[g][src] + adv[g][dst]
        e = jnp.where(e > 0, e, 0.2 * e)
        ee = jnp.exp(e)
        msg = h_aug[g][src] * ee[:, None]
        acc = jax.ops.segment_sum(msg, dst, num_segments=NP)
        accs.append(acc)
    return jnp.stack(accs)


def _k4_body(acc_ref, b1_ref, w2_ref, as2_ref, ad2_ref,
             h2aug_ref, asout_ref, adout_ref):
    i = pl.program_id(0)
    rows = jax.lax.broadcasted_iota(jnp.int32, (BR, 1), 0) + i * BR
    valid = (rows < N).astype(jnp.float32)
    xs = []
    for g in range(H1):
        num = acc_ref[g, :, 0:UNITS]
        den = acc_ref[g, :, UNITS:UNITS + 1] + 1e-9
        xg = num / den + b1_ref[0, g * UNITS:(g + 1) * UNITS][None, :]
        xs.append(jnp.maximum(xg, 0.0) * valid)
    x = jnp.concatenate(xs, axis=1)
    h = jnp.dot(x, w2_ref[...], preferred_element_type=jnp.float32)
    asv = jnp.sum(h * as2_ref[...], axis=-1, keepdims=True)
    adv = jnp.sum(h * ad2_ref[...], axis=-1, keepdims=True)
    zpad = jnp.zeros((BR, 7), jnp.float32)
    asout_ref[...] = jnp.concatenate([asv * valid, zpad], axis=1)
    adout_ref[...] = jnp.concatenate([adv * valid, zpad], axis=1)
    ones = valid
    zc = jnp.zeros((BR, CROW - UNITS - 1), jnp.float32)
    h2aug_ref[...] = jnp.concatenate([h * valid, ones, zc], axis=1)


def _epi1_front2(acc1, b1, W2, a_src2, a_dst2):
    grid = NP // BR
    return pl.pallas_call(
        _k4_body,
        grid=(grid,),
        in_specs=[
            pl.BlockSpec((H1, BR, CROW), lambda i: (0, i, 0)),
            pl.BlockSpec((1, H1 * UNITS), lambda i: (0, 0)),
            pl.BlockSpec((H1 * UNITS, UNITS), lambda i: (0, 0)),
            pl.BlockSpec((1, UNITS), lambda i: (0, 0)),
            pl.BlockSpec((1, UNITS), lambda i: (0, 0)),
        ],
        out_specs=[
            pl.BlockSpec((BR, CROW), lambda i: (i, 0)),
            pl.BlockSpec((BR, 8), lambda i: (i, 0)),
            pl.BlockSpec((BR, 8), lambda i: (i, 0)),
        ],
        out_shape=[
            jax.ShapeDtypeStruct((NP, CROW), jnp.float32),
            jax.ShapeDtypeStruct((NP, 8), jnp.float32),
            jax.ShapeDtypeStruct((NP, 8), jnp.float32),
        ],
    )(acc1, b1, W2, a_src2, a_dst2)


def _k6_body(acc_ref, b2_ref, out_ref):
    num = acc_ref[0, :, 0:UNITS]
    den = acc_ref[0, :, UNITS:UNITS + 1] + 1e-9
    out_ref[...] = num / den + b2_ref[...]


def _epi2(acc2, b2):
    grid = NP // BR
    return pl.pallas_call(
        _k6_body,
        grid=(grid,),
        in_specs=[
            pl.BlockSpec((1, BR, CROW), lambda i: (0, i, 0)),
            pl.BlockSpec((1, UNITS), lambda i: (0, 0)),
        ],
        out_specs=pl.BlockSpec((BR, UNITS), lambda i: (i, 0)),
        out_shape=jax.ShapeDtypeStruct((NP, UNITS), jnp.float32),
    )(acc2, b2)


def kernel(inputs, node_embeddings, W1, a_src1, a_dst1, b1,
           W2, a_src2, a_dst2, b2):
    x_pad = jnp.zeros((NP, D), jnp.float32).at[:N].set(inputs)
    emb_pad = jnp.zeros((NP, EMB), jnp.float32).at[:N].set(node_embeddings)

    h1aug, nemb, as1, ad1 = _dense_front(x_pad, emb_pad, W1, a_src1, a_dst1)
    dst2d, src2d = _topk_edges(nemb, nemb.T)
    dst = dst2d.reshape(-1)
    src = src2d.reshape(-1)

    as1t = as1.T
    ad1t = ad1.T
    acc1 = _seg_aggregate(h1aug, as1t, ad1t, dst, src, H1)

    h2aug, as2, ad2 = _epi1_front2(acc1, b1.reshape(1, -1), W2, a_src2, a_dst2)
    acc2 = _seg_aggregate(h2aug[None], as2.T, ad2.T, dst, src, 1)

    out = _epi2(acc2, b2.reshape(1, -1))
    return out[:N]


# speedup vs baseline: 20.0624x; 15.4536x over previous
import functools
import jax
import jax.numpy as jnp
from jax import lax
from jax.experimental import pallas as pl
from jax.experimental.pallas import tpu as pltpu
from jax.experimental.pallas import tpu_sc as plsc

N = 10000
NP = 10240
D = 256
UNITS = 128
H1 = 4
K = 20
EMB = 16
CROW = 144
EPR = 24
BR = 128
NEG = -3e38


def _k1_body(x_ref, emb_ref, w1_ref, as1_ref, ad1_ref,
             h1aug_ref, nemb_ref, asout_ref, adout_ref):
    i = pl.program_id(0)
    rows = jax.lax.broadcasted_iota(jnp.int32, (BR, 1), 0) + i * BR
    valid = (rows < N).astype(jnp.float32)

    e = emb_ref[...]
    nrm = jnp.sqrt(jnp.maximum(jnp.sum(e * e, axis=-1, keepdims=True), 1e-12))
    nemb_ref[...] = e / nrm

    h = jnp.dot(x_ref[...], w1_ref[...],
                preferred_element_type=jnp.float32)
    h3 = h.reshape(BR, H1, UNITS)
    asv = jnp.sum(h3 * as1_ref[...][None, :, :], axis=-1)
    adv = jnp.sum(h3 * ad1_ref[...][None, :, :], axis=-1)
    zpad = jnp.zeros((BR, 8 - H1), jnp.float32)
    asout_ref[...] = jnp.concatenate([asv * valid, zpad], axis=1)
    adout_ref[...] = jnp.concatenate([adv * valid, zpad], axis=1)

    ones = valid
    zc = jnp.zeros((BR, CROW - UNITS - 1), jnp.float32)
    for g in range(H1):
        blk = jnp.concatenate([h3[:, g, :] * valid, ones, zc], axis=1)
        h1aug_ref[g, :, :] = blk


def _dense_front(x_pad, emb_pad, W1, a_src1, a_dst1):
    grid = NP // BR
    return pl.pallas_call(
        _k1_body,
        grid=(grid,),
        in_specs=[
            pl.BlockSpec((BR, D), lambda i: (i, 0)),
            pl.BlockSpec((BR, EMB), lambda i: (i, 0)),
            pl.BlockSpec((D, H1 * UNITS), lambda i: (0, 0)),
            pl.BlockSpec((H1, UNITS), lambda i: (0, 0)),
            pl.BlockSpec((H1, UNITS), lambda i: (0, 0)),
        ],
        out_specs=[
            pl.BlockSpec((H1, BR, CROW), lambda i: (0, i, 0)),
            pl.BlockSpec((BR, EMB), lambda i: (i, 0)),
            pl.BlockSpec((BR, 8), lambda i: (i, 0)),
            pl.BlockSpec((BR, 8), lambda i: (i, 0)),
        ],
        out_shape=[
            jax.ShapeDtypeStruct((H1, NP, CROW), jnp.float32),
            jax.ShapeDtypeStruct((NP, EMB), jnp.float32),
            jax.ShapeDtypeStruct((NP, 8), jnp.float32),
            jax.ShapeDtypeStruct((NP, 8), jnp.float32),
        ],
    )(x_pad, emb_pad, W1, a_src1, a_dst1)


NCHUNK = NP // 128


def _topk_body(nemb_ref, nembT_ref, dst_ref, src_ref):
    i = pl.program_id(0)
    sim = jnp.dot(nemb_ref[...], nembT_ref[...],
                  preferred_element_type=jnp.float32)
    col = jax.lax.broadcasted_iota(jnp.int32, (BR, NP), 1)
    sim = jnp.where(col < N, sim, NEG)

    lane = jax.lax.broadcasted_iota(jnp.int32, (BR, 128), 1)

    def fold(c, carry):
        m1, m2, m3, m4, c1, c2, c3, c4 = carry
        v = sim[:, c * 128:(c + 1) * 128]
        ci = jnp.full((BR, 128), c, jnp.int32)
        g1 = v > m1
        g2 = v > m2
        g3 = v > m3
        g4 = v > m4
        n1 = jnp.where(g1, v, m1)
        n2 = jnp.where(g1, m1, jnp.where(g2, v, m2))
        n3 = jnp.where(g2, m2, jnp.where(g3, v, m3))
        n4 = jnp.where(g3, m3, jnp.where(g4, v, m4))
        i1 = jnp.where(g1, ci, c1)
        i2 = jnp.where(g1, c1, jnp.where(g2, ci, c2))
        i3 = jnp.where(g2, c2, jnp.where(g3, ci, c3))
        i4 = jnp.where(g3, c3, jnp.where(g4, ci, c4))
        return n1, n2, n3, n4, i1, i2, i3, i4

    neg = jnp.full((BR, 128), NEG, jnp.float32)
    zi = jnp.zeros((BR, 128), jnp.int32)
    carry = (neg, neg, neg, neg, zi, zi, zi, zi)
    for c in range(NCHUNK):
        carry = fold(c, carry)
    m1, m2, m3, m4, c1, c2, c3, c4 = carry

    rows = jax.lax.broadcasted_iota(jnp.int32, (BR, 1), 0) + i * BR
    lane24 = jax.lax.broadcasted_iota(jnp.int32, (BR, EPR), 1)
    BIGI = jnp.int32(2 ** 30)

    def extract(t, carry):
        m1, m2, m3, m4, out = carry
        cur = jnp.maximum(jnp.maximum(m1, m2), jnp.maximum(m3, m4))
        rmax = jnp.max(cur, axis=1, keepdims=True)
        k1 = jnp.where(m1 >= rmax, c1 * 128 + lane, BIGI)
        k2 = jnp.where(m2 >= rmax, c2 * 128 + lane, BIGI)
        k3 = jnp.where(m3 >= rmax, c3 * 128 + lane, BIGI)
        k4 = jnp.where(m4 >= rmax, c4 * 128 + lane, BIGI)
        kk = jnp.minimum(jnp.minimum(k1, k2), jnp.minimum(k3, k4))
        idx = jnp.min(kk, axis=1, keepdims=True)
        out = jnp.where(lane24 == t, idx, out)
        m1 = jnp.where(k1 == idx, NEG, m1)
        m2 = jnp.where((k2 == idx) & (k1 != idx), NEG, m2)
        m3 = jnp.where((k3 == idx) & (k2 != idx) & (k1 != idx), NEG, m3)
        m4 = jnp.where((k4 == idx) & (k3 != idx) & (k2 != idx) & (k1 != idx),
                       NEG, m4)
        return m1, m2, m3, m4, out

    out0 = jnp.zeros((BR, EPR), jnp.int32)
    _, _, _, _, out = lax.fori_loop(0, K, extract, (m1, m2, m3, m4, out0))

    dst_ref[...] = jnp.where(lane24 >= K, rows, out)
    src_ref[...] = jnp.where(lane24 <= K, rows, jnp.int32(NP - 1))


def _topk_edges(nemb, nembT):
    grid = NP // BR
    return pl.pallas_call(
        _topk_body,
        grid=(grid,),
        in_specs=[
            pl.BlockSpec((BR, EMB), lambda i: (i, 0)),
            pl.BlockSpec((EMB, NP), lambda i: (0, 0)),
        ],
        out_specs=[
            pl.BlockSpec((BR, EPR), lambda i: (i, 0)),
            pl.BlockSpec((BR, EPR), lambda i: (i, 0)),
        ],
        out_shape=[
            jax.ShapeDtypeStruct((NP, EPR), jnp.int32),
            jax.ShapeDtypeStruct((NP, EPR), jnp.int32),
        ],
    )(nemb, nembT)


RPC = 4
ECH = RPC * EPR
NVEC = ECH // 16
NQ = CROW // 16
ROWS_T1 = NP // 16
ROWS_T2 = NP // 32


def _sc_zero_and_cols(g, row0, rows_t, acc_sh, as_col, ad_col, zbuf,
                      ast, adt):
    for r in range(16):
        for q in range(NQ):
            zbuf[r, q * 16:(q + 1) * 16] = jnp.zeros((16,), jnp.float32)

    def zloop(j, _):
        pltpu.sync_copy(zbuf, acc_sh.at[pl.ds(row0 + 16 * j, 16)])
        return 0
    lax.fori_loop(0, rows_t // 16, zloop, 0)
    pltpu.sync_copy(ast.at[g], as_col)
    pltpu.sync_copy(adt.at[g], ad_col)


def _sc_edge_sweep(haug_g, dstr, srcr, row0, rows_t, acc_sh,
                   as_col, ad_col, hrow, msg, idxb, srcb, eeb):
    def chunk(ch, _):
        rbase = row0 + ch * RPC
        ebase = rbase * EPR
        pltpu.sync_copy(dstr.at[pl.ds(ebase, ECH)], idxb)
        pltpu.sync_copy(srcr.at[pl.ds(ebase, ECH)], srcb)
        pltpu.sync_copy(haug_g.at[pl.ds(rbase, RPC)], hrow)
        eevs = []
        for v in range(NVEC):
            sv = srcb[pl.ds(v * 16, 16)]
            dv = idxb[pl.ds(v * 16, 16)]
            a = (plsc.load_gather(as_col, [sv])
                 + plsc.load_gather(ad_col, [dv]))
            a = jnp.where(a > 0, a, 0.2 * a)
            eevs.append(jnp.where(sv == NP - 1, 0.0, jnp.exp(a)))
        onehot = [(jax.lax.broadcasted_iota(jnp.int32, (16,), 0) == j)
                  .astype(jnp.float32) for j in range(16)]
        for r in range(RPC):
            hr = [hrow[r, q * 16:(q + 1) * 16] for q in range(NQ)]
            for j in range(EPR):
                m = r * EPR + j
                b = jnp.sum(eevs[m // 16] * onehot[m % 16])
                for q in range(NQ):
                    msg[m, q * 16:(q + 1) * 16] = hr[q] * b
        pltpu.sync_copy(msg, acc_sh.at[idxb], add=True)
        return 0
    lax.fori_loop(0, rows_t // RPC, chunk, 0)


def _sc_agg1(h1aug, ast, adt, dst, src):
    mesh = plsc.VectorSubcoreMesh(core_axis_name="c", subcore_axis_name="s")

    @functools.partial(
        pl.kernel, mesh=mesh,
        compiler_params=pltpu.CompilerParams(needs_layout_passes=False, use_tc_tiling_on_sc=False),
        out_type=jax.ShapeDtypeStruct((H1, NP, CROW), jnp.float32),
        scratch_types=[
            pltpu.VMEM_SHARED((NP, CROW), jnp.float32),
            pltpu.VMEM((NP,), jnp.float32),
            pltpu.VMEM((NP,), jnp.float32),
            pltpu.VMEM((RPC, CROW), jnp.float32),
            pltpu.VMEM((ECH, CROW), jnp.float32),
            pltpu.VMEM((ECH,), jnp.int32),
            pltpu.VMEM((ECH,), jnp.int32),
            pltpu.VMEM((ECH,), jnp.float32),
            pltpu.VMEM((16, CROW), jnp.float32),
        ],
    )
    def k(h1aug_r, ast_r, adt_r, dst_r, src_r, out_r,
          acc_sh, as_col, ad_col, hrow, msg, idxb, srcb, eeb, zbuf):
        c = lax.axis_index("c")
        s = lax.axis_index("s")
        row0 = s * ROWS_T1
        for sweep in range(2):
            g = c * 2 + sweep
            _sc_zero_and_cols(g, row0, ROWS_T1, acc_sh, as_col, ad_col,
                              zbuf, ast_r, adt_r)
            plsc.subcore_barrier()
            _sc_edge_sweep(h1aug_r.at[g], dst_r, src_r, row0, ROWS_T1,
                           acc_sh, as_col, ad_col, hrow, msg, idxb, srcb, eeb)
            plsc.subcore_barrier()
            pltpu.sync_copy(acc_sh.at[pl.ds(row0, ROWS_T1)],
                            out_r.at[g].at[pl.ds(row0, ROWS_T1)])
            plsc.subcore_barrier()

    return k(h1aug, ast, adt, dst, src)


def _sc_agg2(h2aug, ast, adt, dst, src):
    mesh = plsc.VectorSubcoreMesh(core_axis_name="c", subcore_axis_name="s")

    @functools.partial(
        pl.kernel, mesh=mesh,
        compiler_params=pltpu.CompilerParams(needs_layout_passes=False, use_tc_tiling_on_sc=False),
        out_type=jax.ShapeDtypeStruct((2, NP, CROW), jnp.float32),
        scratch_types=[
            pltpu.VMEM_SHARED((NP, CROW), jnp.float32),
            pltpu.VMEM((NP,), jnp.float32),
            pltpu.VMEM((NP,), jnp.float32),
            pltpu.VMEM((RPC, CROW), jnp.float32),
            pltpu.VMEM((ECH, CROW), jnp.float32),
            pltpu.VMEM((ECH,), jnp.int32),
            pltpu.VMEM((ECH,), jnp.int32),
            pltpu.VMEM((ECH,), jnp.float32),
            pltpu.VMEM((16, CROW), jnp.float32),
        ],
    )
    def k(h2aug_r, ast_r, adt_r, dst_r, src_r, out_r,
          acc_sh, as_col, ad_col, hrow, msg, idxb, srcb, eeb, zbuf):
        c = lax.axis_index("c")
        s = lax.axis_index("s")
        zrow0 = s * ROWS_T1
        erow0 = (c * 16 + s) * ROWS_T2
        _sc_zero_and_cols(0, zrow0, ROWS_T1, acc_sh, as_col, ad_col,
                          zbuf, ast_r, adt_r)
        plsc.subcore_barrier()
        _sc_edge_sweep(h2aug_r, dst_r, src_r, erow0, ROWS_T2,
                       acc_sh, as_col, ad_col, hrow, msg, idxb, srcb, eeb)
        plsc.subcore_barrier()
        pltpu.sync_copy(acc_sh.at[pl.ds(zrow0, ROWS_T1)],
                        out_r.at[c].at[pl.ds(zrow0, ROWS_T1)])

    return k(h2aug, ast, adt, dst, src)


def _k4_body(acc_ref, b1_ref, w2_ref, as2_ref, ad2_ref,
             h2aug_ref, asout_ref, adout_ref):
    i = pl.program_id(0)
    rows = jax.lax.broadcasted_iota(jnp.int32, (BR, 1), 0) + i * BR
    valid = (rows < N).astype(jnp.float32)
    xs = []
    for g in range(H1):
        num = acc_ref[g, :, 0:UNITS]
        den = acc_ref[g, :, UNITS:UNITS + 1] + 1e-9
        xg = num / den + b1_ref[0, g * UNITS:(g + 1) * UNITS][None, :]
        xs.append(jnp.maximum(xg, 0.0) * valid)
    x = jnp.concatenate(xs, axis=1)
    h = jnp.dot(x, w2_ref[...], preferred_element_type=jnp.float32)
    asv = jnp.sum(h * as2_ref[...], axis=-1, keepdims=True)
    adv = jnp.sum(h * ad2_ref[...], axis=-1, keepdims=True)
    zpad = jnp.zeros((BR, 7), jnp.float32)
    asout_ref[...] = jnp.concatenate([asv * valid, zpad], axis=1)
    adout_ref[...] = jnp.concatenate([adv * valid, zpad], axis=1)
    ones = valid
    zc = jnp.zeros((BR, CROW - UNITS - 1), jnp.float32)
    h2aug_ref[...] = jnp.concatenate([h * valid, ones, zc], axis=1)


def _epi1_front2(acc1, b1, W2, a_src2, a_dst2):
    grid = NP // BR
    return pl.pallas_call(
        _k4_body,
        grid=(grid,),
        in_specs=[
            pl.BlockSpec((H1, BR, CROW), lambda i: (0, i, 0)),
            pl.BlockSpec((1, H1 * UNITS), lambda i: (0, 0)),
            pl.BlockSpec((H1 * UNITS, UNITS), lambda i: (0, 0)),
            pl.BlockSpec((1, UNITS), lambda i: (0, 0)),
            pl.BlockSpec((1, UNITS), lambda i: (0, 0)),
        ],
        out_specs=[
            pl.BlockSpec((BR, CROW), lambda i: (i, 0)),
            pl.BlockSpec((BR, 8), lambda i: (i, 0)),
            pl.BlockSpec((BR, 8), lambda i: (i, 0)),
        ],
        out_shape=[
            jax.ShapeDtypeStruct((NP, CROW), jnp.float32),
            jax.ShapeDtypeStruct((NP, 8), jnp.float32),
            jax.ShapeDtypeStruct((NP, 8), jnp.float32),
        ],
    )(acc1, b1, W2, a_src2, a_dst2)


def _k6_body(acc_ref, b2_ref, out_ref):
    num = acc_ref[0, :, 0:UNITS] + acc_ref[1, :, 0:UNITS]
    den = (acc_ref[0, :, UNITS:UNITS + 1]
           + acc_ref[1, :, UNITS:UNITS + 1] + 1e-9)
    out_ref[...] = num / den + b2_ref[...]


def _epi2(acc2, b2):
    grid = NP // BR
    return pl.pallas_call(
        _k6_body,
        grid=(grid,),
        in_specs=[
            pl.BlockSpec((2, BR, CROW), lambda i: (0, i, 0)),
            pl.BlockSpec((1, UNITS), lambda i: (0, 0)),
        ],
        out_specs=pl.BlockSpec((BR, UNITS), lambda i: (i, 0)),
        out_shape=jax.ShapeDtypeStruct((NP, UNITS), jnp.float32),
    )(acc2, b2)


def kernel(inputs, node_embeddings, W1, a_src1, a_dst1, b1,
           W2, a_src2, a_dst2, b2):
    x_pad = jnp.zeros((NP, D), jnp.float32).at[:N].set(inputs)
    emb_pad = jnp.zeros((NP, EMB), jnp.float32).at[:N].set(node_embeddings)

    h1aug, nemb, as1, ad1 = _dense_front(x_pad, emb_pad, W1, a_src1, a_dst1)
    dst2d, src2d = _topk_edges(nemb, nemb.T)
    dst = dst2d.reshape(-1)
    src = src2d.reshape(-1)

    as1t = as1.T + 0.0
    ad1t = ad1.T + 0.0
    acc1 = _sc_agg1(h1aug, as1t, ad1t, dst, src)

    h2aug, as2, ad2 = _epi1_front2(acc1, b1.reshape(1, -1), W2, a_src2, a_dst2)
    acc2 = _sc_agg2(h2aug, as2.T + 0.0,
                    ad2.T + 0.0, dst, src)

    out = _epi2(acc2, b2.reshape(1, -1))
    return out[:N]


# speedup vs baseline: 27.8141x; 1.3864x over previous
import functools
import jax
import jax.numpy as jnp
from jax import lax
from jax.experimental import pallas as pl
from jax.experimental.pallas import tpu as pltpu
from jax.experimental.pallas import tpu_sc as plsc

N = 10000
NP = 10240
D = 256
UNITS = 128
H1 = 4
K = 20
EMB = 16
CROW = 144
EPR = 24
NACC = 10048
TRASH = 10016
BR = 128
NEG = -3e38


def _k1_body(x_ref, emb_ref, w1_ref, as1_ref, ad1_ref,
             h1aug_ref, nemb_ref, asout_ref, adout_ref):
    i = pl.program_id(0)
    rows = jax.lax.broadcasted_iota(jnp.int32, (BR, 1), 0) + i * BR
    valid = (rows < N).astype(jnp.float32)

    e = emb_ref[...]
    nrm = jnp.sqrt(jnp.maximum(jnp.sum(e * e, axis=-1, keepdims=True), 1e-12))
    nemb_ref[...] = e / nrm

    h = jnp.dot(x_ref[...], w1_ref[...],
                preferred_element_type=jnp.float32)
    h3 = h.reshape(BR, H1, UNITS)
    asv = jnp.sum(h3 * as1_ref[...][None, :, :], axis=-1)
    adv = jnp.sum(h3 * ad1_ref[...][None, :, :], axis=-1)
    zpad = jnp.zeros((BR, 8 - H1), jnp.float32)
    asout_ref[...] = jnp.concatenate([asv * valid, zpad], axis=1)
    adout_ref[...] = jnp.concatenate([adv * valid, zpad], axis=1)

    ones = valid
    zc = jnp.zeros((BR, CROW - UNITS - 1), jnp.float32)
    for g in range(H1):
        blk = jnp.concatenate([h3[:, g, :] * valid, ones, zc], axis=1)
        h1aug_ref[g, :, :] = blk


def _dense_front(x_pad, emb_pad, W1, a_src1, a_dst1):
    grid = NP // BR
    return pl.pallas_call(
        _k1_body,
        grid=(grid,),
        in_specs=[
            pl.BlockSpec((BR, D), lambda i: (i, 0)),
            pl.BlockSpec((BR, EMB), lambda i: (i, 0)),
            pl.BlockSpec((D, H1 * UNITS), lambda i: (0, 0)),
            pl.BlockSpec((H1, UNITS), lambda i: (0, 0)),
            pl.BlockSpec((H1, UNITS), lambda i: (0, 0)),
        ],
        out_specs=[
            pl.BlockSpec((H1, BR, CROW), lambda i: (0, i, 0)),
            pl.BlockSpec((BR, EMB), lambda i: (i, 0)),
            pl.BlockSpec((BR, 8), lambda i: (i, 0)),
            pl.BlockSpec((BR, 8), lambda i: (i, 0)),
        ],
        out_shape=[
            jax.ShapeDtypeStruct((H1, NP, CROW), jnp.float32),
            jax.ShapeDtypeStruct((NP, EMB), jnp.float32),
            jax.ShapeDtypeStruct((NP, 8), jnp.float32),
            jax.ShapeDtypeStruct((NP, 8), jnp.float32),
        ],
    )(x_pad, emb_pad, W1, a_src1, a_dst1)


NCHUNK = NP // 128


def _topk_body(nemb_ref, nembT_ref, dst_ref):
    i = pl.program_id(0)
    sim = jnp.dot(nemb_ref[...], nembT_ref[...],
                  preferred_element_type=jnp.float32)
    col = jax.lax.broadcasted_iota(jnp.int32, (BR, NP), 1)
    sim = jnp.where(col < N, sim, NEG)

    lane = jax.lax.broadcasted_iota(jnp.int32, (BR, 128), 1)

    def fold(c, carry):
        m1, m2, m3, m4, c1, c2, c3, c4 = carry
        v = sim[:, c * 128:(c + 1) * 128]
        ci = jnp.full((BR, 128), c, jnp.int32)
        g1 = v > m1
        g2 = v > m2
        g3 = v > m3
        g4 = v > m4
        n1 = jnp.where(g1, v, m1)
        n2 = jnp.where(g1, m1, jnp.where(g2, v, m2))
        n3 = jnp.where(g2, m2, jnp.where(g3, v, m3))
        n4 = jnp.where(g3, m3, jnp.where(g4, v, m4))
        i1 = jnp.where(g1, ci, c1)
        i2 = jnp.where(g1, c1, jnp.where(g2, ci, c2))
        i3 = jnp.where(g2, c2, jnp.where(g3, ci, c3))
        i4 = jnp.where(g3, c3, jnp.where(g4, ci, c4))
        return n1, n2, n3, n4, i1, i2, i3, i4

    neg = jnp.full((BR, 128), NEG, jnp.float32)
    zi = jnp.zeros((BR, 128), jnp.int32)
    carry = (neg, neg, neg, neg, zi, zi, zi, zi)
    for c in range(NCHUNK):
        carry = fold(c, carry)
    m1, m2, m3, m4, c1, c2, c3, c4 = carry

    rows = jax.lax.broadcasted_iota(jnp.int32, (BR, 1), 0) + i * BR
    lane24 = jax.lax.broadcasted_iota(jnp.int32, (BR, EPR), 1)
    BIGI = jnp.int32(2 ** 30)

    def extract(t, carry):
        m1, m2, m3, m4, out = carry
        cur = jnp.maximum(jnp.maximum(m1, m2), jnp.maximum(m3, m4))
        rmax = jnp.max(cur, axis=1, keepdims=True)
        k1 = jnp.where(m1 >= rmax, c1 * 128 + lane, BIGI)
        k2 = jnp.where(m2 >= rmax, c2 * 128 + lane, BIGI)
        k3 = jnp.where(m3 >= rmax, c3 * 128 + lane, BIGI)
        k4 = jnp.where(m4 >= rmax, c4 * 128 + lane, BIGI)
        kk = jnp.minimum(jnp.minimum(k1, k2), jnp.minimum(k3, k4))
        idx = jnp.min(kk, axis=1, keepdims=True)
        out = jnp.where(lane24 == t, idx, out)
        m1 = jnp.where(k1 == idx, NEG, m1)
        m2 = jnp.where((k2 == idx) & (k1 != idx), NEG, m2)
        m3 = jnp.where((k3 == idx) & (k2 != idx) & (k1 != idx), NEG, m3)
        m4 = jnp.where((k4 == idx) & (k3 != idx) & (k2 != idx) & (k1 != idx),
                       NEG, m4)
        return m1, m2, m3, m4, out

    out0 = jnp.zeros((BR, EPR), jnp.int32)
    _, _, _, _, out = lax.fori_loop(0, K, extract, (m1, m2, m3, m4, out0))

    d = jnp.where(lane24 >= K, rows, out)
    d = jnp.where(lane24 > K, jnp.int32(TRASH), d)
    dst_ref[...] = jnp.where(rows < N, d, jnp.int32(TRASH))


def _topk_edges(nemb, nembT):
    grid = NP // BR
    return pl.pallas_call(
        _topk_body,
        grid=(grid,),
        in_specs=[
            pl.BlockSpec((BR, EMB), lambda i: (i, 0)),
            pl.BlockSpec((EMB, NP), lambda i: (0, 0)),
        ],
        out_specs=pl.BlockSpec((BR, EPR), lambda i: (i, 0)),
        out_shape=jax.ShapeDtypeStruct((NP, EPR), jnp.int32),
    )(nemb, nembT)


RPC = 4
ECH = RPC * EPR
NVEC = ECH // 16
NQ = CROW // 16
ROWS_T1 = NP // 16
ROWS_T2 = NP // 32
ZROWS = NACC // 16


def _sc_edge_sweep(haug_g, dstr, ast_g, ad_col, erow0, rows_t, acc_sh,
                   asT, hrow, msg, ldx0, ldx1, sidx0, sidx1, dvw,
                   sem_i, sem_h, sem_s):
    nch = rows_t // RPC
    pltpu.sync_copy(ast_g.at[pl.ds(erow0, rows_t)], asT.at[pl.ds(0, rows_t)])

    def issue_idx(c, ldx):
        pltpu.async_copy(dstr.at[pl.ds((erow0 + c * RPC) * EPR, ECH)],
                         ldx, sem_i)

    def wait_idx(ldx):
        pltpu.make_async_copy(dstr.at[pl.ds(erow0 * EPR, ECH)], ldx,
                              sem_i).wait()

    def issue_hrow(c):
        hoff = (c % 2) * RPC
        pltpu.async_copy(haug_g.at[pl.ds(erow0 + c * RPC, RPC)],
                         hrow.at[pl.ds(hoff, RPC)], sem_h)

    def wait_hrow():
        pltpu.make_async_copy(haug_g.at[pl.ds(erow0, RPC)],
                              hrow.at[pl.ds(0, RPC)], sem_h).wait()

    def issue_scatter(sidx, moff):
        pltpu.async_copy(msg.at[pl.ds(moff, ECH)], acc_sh.at[sidx],
                         sem_s, add=True)

    def wait_scatter(sidx):
        pltpu.make_async_copy(msg.at[pl.ds(0, ECH)], acc_sh.at[sidx],
                              sem_s).wait()

    issue_hrow(0)
    issue_idx(0, ldx0)
    iota16 = jax.lax.broadcasted_iota(jnp.int32, (16,), 0)
    lane = [iota16 * 0 + j for j in range(16)]
    rowpat = [(iota16 + v * 16) // EPR for v in range(NVEC)]

    def chunk(c, _):
        par = c % 2
        moff = par * ECH
        hoff = par * RPC
        wait_hrow()
        wait_idx(ldx0)

        @pl.when(c + 1 < nch)
        def _():
            issue_hrow(c + 1)

        @pl.when(c >= 2)
        def _():
            wait_scatter(sidx0)

        @pl.when(par == 0)
        def _():
            for v in range(NVEC):
                x = ldx0[pl.ds(v * 16, 16)]
                dvw[pl.ds(v * 16, 16)] = x
                sidx0[pl.ds(v * 16, 16)] = x

            @pl.when(c + 1 < nch)
            def _():
                issue_idx(c + 1, ldx1)

        @pl.when(par == 1)
        def _():
            for v in range(NVEC):
                x = ldx1[pl.ds(v * 16, 16)]
                dvw[pl.ds(v * 16, 16)] = x
                sidx1[pl.ds(v * 16, 16)] = x

            @pl.when(c + 1 < nch)
            def _():
                issue_idx(c + 1, ldx0)

        base = c * RPC
        eevs = []
        for v in range(NVEC):
            sv = rowpat[v] + base
            dv = dvw[pl.ds(v * 16, 16)]
            a = (plsc.load_gather(asT, [sv])
                 + plsc.load_gather(ad_col, [dv]))
            a = jnp.where(a > 0, a, 0.2 * a)
            eevs.append(jnp.exp(a))
        for r in range(RPC):
            hr = [hrow[hoff + r, q * 16:(q + 1) * 16] for q in range(NQ)]
            for j in range(EPR):
                m = r * EPR + j
                b = eevs[m // 16][lane[m % 16]]
                for q in range(NQ):
                    msg[moff + m, q * 16:(q + 1) * 16] = hr[q] * b

        @pl.when(par == 0)
        def _():
            issue_scatter(sidx0, moff)

        @pl.when(par == 1)
        def _():
            issue_scatter(sidx1, moff)
        return 0

    lax.fori_loop(0, nch, chunk, 0)
    wait_scatter(sidx0)
    wait_scatter(sidx1)


def _sc_scratch():
    return [
        pltpu.VMEM_SHARED((NACC, CROW), jnp.float32),
        pltpu.VMEM((NP,), jnp.float32),
        pltpu.VMEM((ROWS_T1,), jnp.float32),
        pltpu.VMEM((2 * RPC, CROW), jnp.float32),
        pltpu.VMEM((2 * ECH, CROW), jnp.float32),
        pltpu.VMEM((ECH,), jnp.int32),
        pltpu.VMEM((ECH,), jnp.int32),
        pltpu.VMEM((ECH,), jnp.int32),
        pltpu.VMEM((ECH,), jnp.int32),
        pltpu.VMEM((ECH,), jnp.int32),
        pltpu.SemaphoreType.DMA,
        pltpu.SemaphoreType.DMA,
        pltpu.SemaphoreType.DMA,
    ]


_SC_PARAMS = pltpu.CompilerParams(needs_layout_passes=False,
                                  use_tc_tiling_on_sc=False)


def _sc_agg1(h1aug, ast, adt, dst, zeros_in):
    mesh = plsc.VectorSubcoreMesh(core_axis_name="c", subcore_axis_name="s")

    @functools.partial(
        pl.kernel, mesh=mesh, compiler_params=_SC_PARAMS,
        out_type=jax.ShapeDtypeStruct((H1, NACC, CROW), jnp.float32),
        scratch_types=_sc_scratch(),
    )
    def k(h1aug_r, ast_r, adt_r, dst_r, zeros_r, out_r,
          acc_sh, ad_col, asT, hrow, msg, ldx0, ldx1, sidx0, sidx1, dvw,
          sem_i, sem_h, sem_s):
        c = lax.axis_index("c")
        s = lax.axis_index("s")
        row0 = s * ROWS_T1
        zrow0 = s * ZROWS

        def sweep(sw, _):
            g = c * 2 + sw
            pltpu.sync_copy(zeros_r, acc_sh.at[pl.ds(zrow0, ZROWS)])
            pltpu.sync_copy(adt_r.at[g], ad_col)
            plsc.subcore_barrier()
            _sc_edge_sweep(h1aug_r.at[g], dst_r, ast_r.at[g], ad_col,
                           row0, ROWS_T1, acc_sh, asT, hrow, msg,
                           ldx0, ldx1, sidx0, sidx1, dvw,
                           sem_i, sem_h, sem_s)
            plsc.subcore_barrier()
            pltpu.sync_copy(acc_sh.at[pl.ds(zrow0, ZROWS)],
                            out_r.at[g].at[pl.ds(zrow0, ZROWS)])
            plsc.subcore_barrier()
            return 0

        lax.fori_loop(0, 2, sweep, 0)

    return k(h1aug, ast, adt, dst, zeros_in)


def _sc_agg2(h2aug, ast, adt, dst, zeros_in):
    mesh = plsc.VectorSubcoreMesh(core_axis_name="c", subcore_axis_name="s")

    @functools.partial(
        pl.kernel, mesh=mesh, compiler_params=_SC_PARAMS,
        out_type=jax.ShapeDtypeStruct((2, NACC, CROW), jnp.float32),
        scratch_types=_sc_scratch(),
    )
    def k(h2aug_r, ast_r, adt_r, dst_r, zeros_r, out_r,
          acc_sh, ad_col, asT, hrow, msg, ldx0, ldx1, sidx0, sidx1, dvw,
          sem_i, sem_h, sem_s):
        c = lax.axis_index("c")
        s = lax.axis_index("s")
        zrow0 = s * ZROWS
        erow0 = (c * 16 + s) * ROWS_T2
        pltpu.sync_copy(zeros_r, acc_sh.at[pl.ds(zrow0, ZROWS)])
        pltpu.sync_copy(adt_r.at[0], ad_col)
        plsc.subcore_barrier()
        _sc_edge_sweep(h2aug_r, dst_r, ast_r.at[0], ad_col,
                       erow0, ROWS_T2, acc_sh, asT, hrow, msg,
                       ldx0, ldx1, sidx0, sidx1, dvw, sem_i, sem_h, sem_s)
        plsc.subcore_barrier()
        pltpu.sync_copy(acc_sh.at[pl.ds(zrow0, ZROWS)],
                        out_r.at[c].at[pl.ds(zrow0, ZROWS)])

    return k(h2aug, ast, adt, dst, zeros_in)


def _k4_body(acc_ref, b1_ref, w2_ref, as2_ref, ad2_ref,
             h2aug_ref, asout_ref, adout_ref):
    i = pl.program_id(0)
    rows = jax.lax.broadcasted_iota(jnp.int32, (BR, 1), 0) + i * BR
    valid = (rows < N).astype(jnp.float32)
    xs = []
    for g in range(H1):
        num = acc_ref[g, :, 0:UNITS]
        den = acc_ref[g, :, UNITS:UNITS + 1] + 1e-9
        xg = num / den + b1_ref[0, g * UNITS:(g + 1) * UNITS][None, :]
        xs.append(jnp.maximum(xg, 0.0) * valid)
    x = jnp.concatenate(xs, axis=1)
    h = jnp.dot(x, w2_ref[...], preferred_element_type=jnp.float32)
    asv = jnp.sum(h * as2_ref[...], axis=-1, keepdims=True)
    adv = jnp.sum(h * ad2_ref[...], axis=-1, keepdims=True)
    zpad = jnp.zeros((BR, 7), jnp.float32)
    asout_ref[...] = jnp.concatenate([asv * valid, zpad], axis=1)
    adout_ref[...] = jnp.concatenate([adv * valid, zpad], axis=1)
    ones = valid
    zc = jnp.zeros((BR, CROW - UNITS - 1), jnp.float32)
    h2aug_ref[...] = jnp.concatenate([h * valid, ones, zc], axis=1)


def _epi1_front2(acc1, b1, W2, a_src2, a_dst2):
    grid = NP // BR
    return pl.pallas_call(
        _k4_body,
        grid=(grid,),
        in_specs=[
            pl.BlockSpec((H1, BR, CROW), lambda i: (0, i, 0)),
            pl.BlockSpec((1, H1 * UNITS), lambda i: (0, 0)),
            pl.BlockSpec((H1 * UNITS, UNITS), lambda i: (0, 0)),
            pl.BlockSpec((1, UNITS), lambda i: (0, 0)),
            pl.BlockSpec((1, UNITS), lambda i: (0, 0)),
        ],
        out_specs=[
            pl.BlockSpec((BR, CROW), lambda i: (i, 0)),
            pl.BlockSpec((BR, 8), lambda i: (i, 0)),
            pl.BlockSpec((BR, 8), lambda i: (i, 0)),
        ],
        out_shape=[
            jax.ShapeDtypeStruct((NP, CROW), jnp.float32),
            jax.ShapeDtypeStruct((NP, 8), jnp.float32),
            jax.ShapeDtypeStruct((NP, 8), jnp.float32),
        ],
    )(acc1, b1, W2, a_src2, a_dst2)


def _k6_body(acc_ref, b2_ref, out_ref):
    num = acc_ref[0, :, 0:UNITS] + acc_ref[1, :, 0:UNITS]
    den = (acc_ref[0, :, UNITS:UNITS + 1]
           + acc_ref[1, :, UNITS:UNITS + 1] + 1e-9)
    out_ref[...] = num / den + b2_ref[...]


def _epi2(acc2, b2):
    grid = NP // BR
    return pl.pallas_call(
        _k6_body,
        grid=(grid,),
        in_specs=[
            pl.BlockSpec((2, BR, CROW), lambda i: (0, i, 0)),
            pl.BlockSpec((1, UNITS), lambda i: (0, 0)),
        ],
        out_specs=pl.BlockSpec((BR, UNITS), lambda i: (i, 0)),
        out_shape=jax.ShapeDtypeStruct((NP, UNITS), jnp.float32),
    )(acc2, b2)


def kernel(inputs, node_embeddings, W1, a_src1, a_dst1, b1,
           W2, a_src2, a_dst2, b2):
    x_pad = jnp.zeros((NP, D), jnp.float32).at[:N].set(inputs)
    emb_pad = jnp.zeros((NP, EMB), jnp.float32).at[:N].set(node_embeddings)

    h1aug, nemb, as1, ad1 = _dense_front(x_pad, emb_pad, W1, a_src1, a_dst1)
    dst2d = _topk_edges(nemb, nemb.T)
    dst = dst2d.reshape(-1)
    zeros_in = jnp.zeros((ZROWS, CROW), jnp.float32)

    as1t = as1.T + 0.0
    ad1t = ad1.T + 0.0
    acc1 = _sc_agg1(h1aug, as1t, ad1t, dst, zeros_in)
    acc1 = jnp.pad(acc1, ((0, 0), (0, NP - NACC), (0, 0)))

    h2aug, as2, ad2 = _epi1_front2(acc1, b1.reshape(1, -1), W2, a_src2, a_dst2)
    acc2 = _sc_agg2(h2aug, as2.T + 0.0, ad2.T + 0.0, dst, zeros_in)
    acc2 = jnp.pad(acc2, ((0, 0), (0, NP - NACC), (0, 0)))

    out = _epi2(acc2, b2.reshape(1, -1))
    return out[:N]
